# baseline (speedup 1.0000x reference)
import jax
import jax.numpy as jnp
from jax.experimental import pallas as pl


def _make_mlp_kernel(B, S, nb):
    def _mlp_kernel(x_ref, wr1_ref, bl1_ref, wr2_ref, bl2_ref, w_ref, out_ref):
        for i in range(nb):
            rows = min(S, B - i * S)
            h = jnp.dot(x_ref[i], wr1_ref[...],
                        preferred_element_type=jnp.float32)
            h = jnp.maximum(h + bl1_ref[...], 0.0)
            h = jnp.dot(h, wr2_ref[...], preferred_element_type=jnp.float32)
            h = jnp.maximum(h + bl2_ref[...], 0.0)
            s = jnp.dot(h, w_ref[...], preferred_element_type=jnp.float32)
            out_ref[pl.ds(i * S, rows), :] = s[:rows, :]
    return _mlp_kernel


def kernel(x0, Wl1, bl1, Wr1, Wl2, bl2, Wr2, weight, out_1, out_2):
    B, S, feat = x0.shape
    emb = Wr1.shape[1]
    nc = weight.shape[1]
    nb = -(-B // S)

    return pl.pallas_call(
        _make_mlp_kernel(B, S, nb),
        grid=(1,),
        in_specs=[
            pl.BlockSpec((nb, S, feat), lambda i: (0, 0, 0)),
            pl.BlockSpec((feat, emb), lambda i: (0, 0)),
            pl.BlockSpec((1, emb), lambda i: (0, 0)),
            pl.BlockSpec((emb, emb), lambda i: (0, 0)),
            pl.BlockSpec((1, emb), lambda i: (0, 0)),
            pl.BlockSpec((emb, nc), lambda i: (0, 0)),
        ],
        out_specs=pl.BlockSpec((B, nc), lambda i: (0, 0)),
        out_shape=jax.ShapeDtypeStruct((B, nc), jnp.float32),
    )(x0, Wr1, bl1.reshape(1, emb), Wr2, bl2.reshape(1, emb), weight)


# speedup vs baseline: 15.0511x; 15.0511x over previous
import jax
import jax.numpy as jnp
from jax.experimental import pallas as pl


def _make_mlp_kernel(B, S, nb):
    def _mlp_kernel(x_ref, wr1_ref, bl1_ref, wr2_ref, bl2_ref, w_ref, out_ref):
        for i in range(nb):
            rows = min(S, B - i * S)
            h = jnp.dot(x_ref[i], wr1_ref[...],
                        preferred_element_type=jnp.float32)
            h = jnp.maximum(h + bl1_ref[...], 0.0)
            h = jnp.dot(h, wr2_ref[...], preferred_element_type=jnp.float32)
            h = jnp.maximum(h + bl2_ref[...], 0.0)
            s = jnp.dot(h, w_ref[...], preferred_element_type=jnp.float32)
            out_ref[pl.ds(i * S, rows), :] = s[:rows, :]
    return _mlp_kernel


def kernel(x0, Wl1, bl1, Wr1, Wl2, bl2, Wr2, weight, out_1, out_2):
    B, S, feat = x0.shape
    emb = Wr1.shape[1]
    nc = weight.shape[1]
    nb = -(-B // S)
    x_live = x0[:nb]

    return pl.pallas_call(
        _make_mlp_kernel(B, S, nb),
        grid=(1,),
        in_specs=[
            pl.BlockSpec((nb, S, feat), lambda i: (0, 0, 0)),
            pl.BlockSpec((feat, emb), lambda i: (0, 0)),
            pl.BlockSpec((1, emb), lambda i: (0, 0)),
            pl.BlockSpec((emb, emb), lambda i: (0, 0)),
            pl.BlockSpec((1, emb), lambda i: (0, 0)),
            pl.BlockSpec((emb, nc), lambda i: (0, 0)),
        ],
        out_specs=pl.BlockSpec((B, nc), lambda i: (0, 0)),
        out_shape=jax.ShapeDtypeStruct((B, nc), jnp.float32),
    )(x_live, Wr1, bl1.reshape(1, emb), Wr2, bl2.reshape(1, emb), weight)
